# baseline (speedup 1.0000x reference)
import jax
import jax.numpy as jnp
from jax import lax
from jax.experimental import pallas as pl
from jax.experimental.pallas import tpu as pltpu
from jax.experimental.pallas import tpu_sc as plsc

N = 10000
E = 160000
D = 256
OUT = 256

NC = 2
NS = 16
L = 16

CG = 128
NROW = 10240
SLAB = NROW // NS
C = 128
NCH = 80
EPT = NCH * C
E_PAD = NS * EPT


def _spmm_body(xT, srcE, dstE, wE, y1T, s2T, Q,
               sb0, db0, wb0, rows0, sb1, db1, wb1, rows1,
               esem0, esem1, dsem0, dsem1, gsem0, gsem1, ssem0, ssem1):
  c = lax.axis_index("c")
  s = lax.axis_index("s")
  row0 = s * SLAB
  bufs = ((sb0, db0, wb0, rows0, esem0, dsem0, gsem0, ssem0),
          (sb1, db1, wb1, rows1, esem1, dsem1, gsem1, ssem1))

  def zero_slab():
    @pl.loop(0, C)
    def _(r):
      for j in range(CG // L):
        rows0[r, pl.ds(j * L, L)] = jnp.zeros((L,), jnp.float32)

    for t in range(SLAB // C):
      pltpu.sync_copy(rows0, Q.at[pl.ds(row0 + t * C, C)])

  def sw_issue(i, b):
    sb, _, wb, _, esem, _, _, _ = bufs[b]
    pltpu.async_copy(srcE.at[s, i], sb, esem)
    pltpu.async_copy(wE.at[s, i], wb, esem)

  def sw_wait(i, b):
    sb, _, wb, _, esem, _, _, _ = bufs[b]
    pltpu.make_async_copy(srcE.at[s, i], sb, esem).wait()
    pltpu.make_async_copy(wE.at[s, i], wb, esem).wait()

  def dst_issue(i, b):
    _, db, _, _, _, dsem, _, _ = bufs[b]
    pltpu.async_copy(dstE.at[s, i], db, dsem)

  def dst_wait(i, b):
    _, db, _, _, _, dsem, _, _ = bufs[b]
    pltpu.make_async_copy(dstE.at[s, i], db, dsem).wait()

  def scale(b):
    _, _, wb, rows, _, _, _, _ = bufs[b]

    @pl.loop(0, C // L)
    def _(k):
      w16 = wb[pl.ds(k * L, L)]
      for e in range(L):
        we = w16[e]
        r = k * L + e
        for j in range(CG // L):
          rows[r, pl.ds(j * L, L)] = rows[r, pl.ds(j * L, L)] * we

  def spmm_pass(src_view):
    def gather_issue(b):
      sb, _, _, rows, _, _, gsem, _ = bufs[b]
      pltpu.async_copy(src_view.at[sb], rows, gsem)

    def gather_wait(b):
      sb, _, _, rows, _, _, gsem, _ = bufs[b]
      pltpu.make_async_copy(src_view.at[sb], rows, gsem).wait()

    def scatter_issue(b):
      _, db, _, rows, _, _, _, ssem = bufs[b]
      pltpu.async_copy(rows, Q.at[db], ssem, add=True)

    def scatter_wait(b):
      _, db, _, rows, _, _, _, ssem = bufs[b]
      pltpu.make_async_copy(rows, Q.at[db], ssem).wait()

    def step(i, b, first, next_gather, next_edges):
      gather_wait(b)
      scale(b)
      dst_wait(i, b)
      scatter_issue(b)
      if next_gather:
        if not first:
          scatter_wait(1 - b)
        sw_wait(i + 1, 1 - b)
        gather_issue(1 - b)
        dst_issue(i + 1, 1 - b)
      if next_edges:
        sw_issue(i + 2, b)

    sw_issue(0, 0)
    dst_issue(0, 0)
    sw_wait(0, 0)
    gather_issue(0)
    sw_issue(1, 1)
    step(0, 0, True, True, True)
    step(1, 1, False, True, True)

    @pl.loop(1, NCH // 2 - 1)
    def _(t):
      i0 = 2 * t
      step(i0, 0, False, True, True)
      step(i0 + 1, 1, False, True, True)

    step(NCH - 2, 0, False, True, False)
    step(NCH - 1, 1, False, False, False)
    scatter_wait(0)
    scatter_wait(1)

  zero_slab()
  plsc.subcore_barrier()

  spmm_pass(xT.at[c])
  plsc.subcore_barrier()

  pltpu.sync_copy(Q.at[pl.ds(row0, SLAB)], y1T.at[c, pl.ds(row0, SLAB)])
  zero_slab()
  plsc.subcore_barrier()

  spmm_pass(y1T.at[c])
  plsc.subcore_barrier()

  pltpu.sync_copy(Q.at[pl.ds(row0, SLAB)], s2T.at[c, pl.ds(row0, SLAB)])


def _make_spmm(interpret=False):
  return pl.kernel(
      _spmm_body,
      out_type=(
          jax.ShapeDtypeStruct((NC, NROW, CG), jnp.float32),
          jax.ShapeDtypeStruct((NC, NROW, CG), jnp.float32),
      ),
      mesh=plsc.VectorSubcoreMesh(
          core_axis_name="c", subcore_axis_name="s",
          num_cores=NC, num_subcores=NS),
      scratch_types=[
          pltpu.VMEM_SHARED((NROW, CG), jnp.float32),
          pltpu.VMEM((C,), jnp.int32),
          pltpu.VMEM((C,), jnp.int32),
          pltpu.VMEM((C,), jnp.float32),
          pltpu.VMEM((C, CG), jnp.float32),
          pltpu.VMEM((C,), jnp.int32),
          pltpu.VMEM((C,), jnp.int32),
          pltpu.VMEM((C,), jnp.float32),
          pltpu.VMEM((C, CG), jnp.float32),
          pltpu.SemaphoreType.DMA,
          pltpu.SemaphoreType.DMA,
          pltpu.SemaphoreType.DMA,
          pltpu.SemaphoreType.DMA,
          pltpu.SemaphoreType.DMA,
          pltpu.SemaphoreType.DMA,
          pltpu.SemaphoreType.DMA,
          pltpu.SemaphoreType.DMA,
      ],
      interpret=interpret,
  )


_spmm = _make_spmm()


MMR = 400


def _mm_body(x0b, y1b, s2b, w0, w1, w2, bb, ob):
  a = jnp.dot(x0b[...], w0[...] - w2[...], preferred_element_type=jnp.float32)
  acc2 = None
  for q in range(NC):
    w1s = w1[q * CG:(q + 1) * CG, :]
    w2s = w2[q * CG:(q + 1) * CG, :]
    a = a + jnp.dot(y1b[q], w1s, preferred_element_type=jnp.float32)
    d2 = jnp.dot(s2b[q], w2s, preferred_element_type=jnp.float32)
    acc2 = d2 if acc2 is None else acc2 + d2
  ob[...] = a + 2.0 * acc2 + bb[...]


def _make_mm(interpret=False):
  return pl.pallas_call(
      _mm_body,
      grid=(N // MMR,),
      in_specs=[
          pl.BlockSpec((MMR, D), lambda i: (i, 0)),
          pl.BlockSpec((NC, MMR, CG), lambda i: (0, i, 0)),
          pl.BlockSpec((NC, MMR, CG), lambda i: (0, i, 0)),
          pl.BlockSpec((D, OUT), lambda i: (0, 0)),
          pl.BlockSpec((D, OUT), lambda i: (0, 0)),
          pl.BlockSpec((D, OUT), lambda i: (0, 0)),
          pl.BlockSpec((1, OUT), lambda i: (0, 0)),
      ],
      out_specs=pl.BlockSpec((MMR, OUT), lambda i: (i, 0)),
      out_shape=jax.ShapeDtypeStruct((N, OUT), jnp.float32),
      interpret=interpret,
  )


_mm = _make_mm()


@jax.jit
def kernel(features, edge_index, edge_weight, W, bias):
  x0 = features.reshape(N, D)
  xT = x0.reshape(N, NC, CG).transpose(1, 0, 2)
  xT = jnp.pad(xT, ((0, 0), (0, NROW - N), (0, 0)))

  pad = E_PAD - E
  iot = jnp.arange(pad, dtype=jnp.int32)
  srcp = jnp.concatenate([edge_index[0], iot % N]).reshape(NS, NCH, C)
  dstp = jnp.concatenate([edge_index[1], N + iot % (NROW - N)]).reshape(
      NS, NCH, C)
  wp = jnp.concatenate(
      [edge_weight, jnp.zeros((pad,), jnp.float32)]).reshape(NS, NCH, C)
  y1T, s2T = _spmm(xT, srcp, dstp, wp)

  W3 = W.reshape(D, 3, OUT)
  out = _mm(x0, y1T, s2T, W3[:, 0, :], W3[:, 1, :], W3[:, 2, :],
            bias.reshape(1, OUT))
  return out.reshape(1, N, OUT)


# speedup vs baseline: 1.2770x; 1.2770x over previous
import jax
import jax.numpy as jnp
from jax import lax
from jax.experimental import pallas as pl
from jax.experimental.pallas import tpu as pltpu
from jax.experimental.pallas import tpu_sc as plsc

N = 10000
E = 160000
D = 256
OUT = 256

NC = 2
NS = 16
L = 16

CG = 128
NROW = 10240
SLAB = NROW // NS
C = 128
NCH = 80
EPT = NCH * C
E_PAD = NS * EPT


def _spmm_body(xV, ipkE, wE, y1T, s2T, Q,
               eb0, wb0, rows0, dcp0, eb1, wb1, rows1, dcp1,
               esem0, esem1, gsem0, gsem1, ssem0, ssem1):
  c = lax.axis_index("c")
  s = lax.axis_index("s")
  row0 = s * SLAB
  bufs = ((eb0, wb0, rows0, esem0, gsem0),
          (eb1, wb1, rows1, esem1, gsem1))
  dcps = (dcp0, dcp1)
  ssems = (ssem0, ssem1)

  def zero_slab():
    @pl.loop(0, C)
    def _(r):
      for j in range(CG // L):
        rows0[r, pl.ds(j * L, L)] = jnp.zeros((L,), jnp.float32)

    for t in range(SLAB // C):
      pltpu.sync_copy(rows0, Q.at[pl.ds(row0 + t * C, C)])

  def edges_issue(i, b):
    eb, wb, _, esem, _ = bufs[b]
    pltpu.async_copy(ipkE.at[s, i], eb, esem)
    pltpu.async_copy(wE.at[s, i], wb, esem)

  def edges_wait(i, b):
    eb, wb, _, esem, _ = bufs[b]
    pltpu.make_async_copy(ipkE.at[s, i], eb, esem).wait()
    pltpu.make_async_copy(wE.at[s, i], wb, esem).wait()

  def scale(b):
    _, wb, rows, _, _ = bufs[b]

    @pl.loop(0, C // L)
    def _(k):
      w16 = wb[pl.ds(k * L, L)]
      for e in range(L):
        we = w16[e]
        r = k * L + e
        for j in range(CG // L):
          rows[r, pl.ds(j * L, L)] = rows[r, pl.ds(j * L, L)] * we

  def spmm_pass(src_view, fix_idx):
    def idx_fix(b):
      eb, _, _, _, _ = bufs[b]
      if fix_idx:
        for k in range(C // L):
          v = eb[0, pl.ds(k * L, L)]
          eb[0, pl.ds(k * L, L)] = v * 2 + c
    def gather_issue(b):
      eb, _, rows, _, gsem = bufs[b]
      pltpu.async_copy(src_view.at[eb.at[0]], rows, gsem)

    def gather_wait(b):
      eb, _, rows, _, gsem = bufs[b]
      pltpu.make_async_copy(src_view.at[eb.at[0]], rows, gsem).wait()

    def scatter_wait(b):
      rows = bufs[b][2]
      pltpu.make_async_copy(rows, Q.at[dcps[b]], ssems[b]).wait()

    def step(i, b, first, next_gather, next_edges):
      eb, _, rows, _, _ = bufs[b]
      gather_wait(b)
      if next_gather:
        edges_wait(i + 1, 1 - b)
        idx_fix(1 - b)
        if not first:
          scatter_wait(1 - b)
        gather_issue(1 - b)
      scale(b)
      for k in range(C // L):
        dcps[b][pl.ds(k * L, L)] = eb[1, pl.ds(k * L, L)]
      pltpu.async_copy(rows, Q.at[dcps[b]], ssems[b], add=True)
      if next_edges:
        edges_issue(i + 2, b)

    edges_issue(0, 0)
    edges_wait(0, 0)
    idx_fix(0)
    gather_issue(0)
    edges_issue(1, 1)
    step(0, 0, True, True, True)
    step(1, 1, False, True, True)

    @pl.loop(1, NCH // 2 - 1)
    def _(t):
      i0 = 2 * t
      step(i0, 0, False, True, True)
      step(i0 + 1, 1, False, True, True)

    step(NCH - 2, 0, False, True, False)
    step(NCH - 1, 1, False, False, False)
    scatter_wait(0)
    scatter_wait(1)

  zero_slab()
  plsc.subcore_barrier()

  spmm_pass(xV, True)
  plsc.subcore_barrier()

  pltpu.sync_copy(Q.at[pl.ds(row0, SLAB)], y1T.at[c, pl.ds(row0, SLAB)])
  zero_slab()
  plsc.subcore_barrier()

  spmm_pass(y1T.at[c], False)
  plsc.subcore_barrier()

  pltpu.sync_copy(Q.at[pl.ds(row0, SLAB)], s2T.at[c, pl.ds(row0, SLAB)])


def _make_spmm(interpret=False):
  return pl.kernel(
      _spmm_body,
      out_type=(
          jax.ShapeDtypeStruct((NC, NROW, CG), jnp.float32),
          jax.ShapeDtypeStruct((NC, NROW, CG), jnp.float32),
      ),
      mesh=plsc.VectorSubcoreMesh(
          core_axis_name="c", subcore_axis_name="s",
          num_cores=NC, num_subcores=NS),
      scratch_types=[
          pltpu.VMEM_SHARED((NROW, CG), jnp.float32),
          pltpu.VMEM((2, C), jnp.int32),
          pltpu.VMEM((C,), jnp.float32),
          pltpu.VMEM((C, CG), jnp.float32),
          pltpu.VMEM((C,), jnp.int32),
          pltpu.VMEM((2, C), jnp.int32),
          pltpu.VMEM((C,), jnp.float32),
          pltpu.VMEM((C, CG), jnp.float32),
          pltpu.VMEM((C,), jnp.int32),
          pltpu.SemaphoreType.DMA,
          pltpu.SemaphoreType.DMA,
          pltpu.SemaphoreType.DMA,
          pltpu.SemaphoreType.DMA,
          pltpu.SemaphoreType.DMA,
          pltpu.SemaphoreType.DMA,
      ],
      interpret=interpret,
  )


_spmm = _make_spmm()


MMR = 400


def _mm_body(x0b, y1b, s2b, w0, w1, w2, bb, ob):
  a = jnp.dot(x0b[...], w0[...] - w2[...], preferred_element_type=jnp.float32)
  acc2 = None
  for q in range(NC):
    w1s = w1[q * CG:(q + 1) * CG, :]
    w2s = w2[q * CG:(q + 1) * CG, :]
    a = a + jnp.dot(y1b[q], w1s, preferred_element_type=jnp.float32)
    d2 = jnp.dot(s2b[q], w2s, preferred_element_type=jnp.float32)
    acc2 = d2 if acc2 is None else acc2 + d2
  ob[...] = a + 2.0 * acc2 + bb[...]


def _make_mm(interpret=False):
  return pl.pallas_call(
      _mm_body,
      grid=(N // MMR,),
      in_specs=[
          pl.BlockSpec((MMR, D), lambda i: (i, 0)),
          pl.BlockSpec((NC, MMR, CG), lambda i: (0, i, 0)),
          pl.BlockSpec((NC, MMR, CG), lambda i: (0, i, 0)),
          pl.BlockSpec((D, OUT), lambda i: (0, 0)),
          pl.BlockSpec((D, OUT), lambda i: (0, 0)),
          pl.BlockSpec((D, OUT), lambda i: (0, 0)),
          pl.BlockSpec((1, OUT), lambda i: (0, 0)),
      ],
      out_specs=pl.BlockSpec((MMR, OUT), lambda i: (i, 0)),
      out_shape=jax.ShapeDtypeStruct((N, OUT), jnp.float32),
      interpret=interpret,
  )


_mm = _make_mm()


@jax.jit
def kernel(features, edge_index, edge_weight, W, bias):
  x0 = features.reshape(N, D)
  xV = x0.reshape(N * NC, CG)

  pad = E_PAD - E
  iot = jnp.arange(pad, dtype=jnp.int32)
  srcp = jnp.concatenate([edge_index[0], iot % N]).reshape(NS, NCH, C)
  dstp = jnp.concatenate([edge_index[1], N + iot % (NROW - N)]).reshape(
      NS, NCH, C)
  wp = jnp.concatenate(
      [edge_weight, jnp.zeros((pad,), jnp.float32)]).reshape(NS, NCH, C)
  ipk = jnp.stack([srcp, dstp], axis=2)
  y1T, s2T = _spmm(xV, ipk, wp)

  W3 = W.reshape(D, 3, OUT)
  out = _mm(x0, y1T, s2T, W3[:, 0, :], W3[:, 1, :], W3[:, 2, :],
            bias.reshape(1, OUT))
  return out.reshape(1, N, OUT)


# speedup vs baseline: 1.2801x; 1.0025x over previous
import jax
import jax.numpy as jnp
from jax import lax
from jax.experimental import pallas as pl
from jax.experimental.pallas import tpu as pltpu
from jax.experimental.pallas import tpu_sc as plsc

N = 10000
E = 160000
D = 256
OUT = 256

NC = 2
NS = 16
L = 16

CG = 128
NROW = 10240
SLAB = NROW // NS
C = 128
NCH = 80
EPT = NCH * C
E_PAD = NS * EPT


def _spmm_body(xV, ipkE, wE, y1T, s2T, Q,
               eb0, wb0, rows0, dcp0, eb1, wb1, rows1, dcp1,
               esem0, esem1, gsem0, gsem1, ssem0, ssem1):
  c = lax.axis_index("c")
  s = lax.axis_index("s")
  row0 = s * SLAB
  bufs = ((eb0, wb0, rows0, esem0, gsem0),
          (eb1, wb1, rows1, esem1, gsem1))
  dcps = (dcp0, dcp1)
  ssems = (ssem0, ssem1)

  def zero_slab():
    @pl.loop(0, C)
    def _(r):
      for j in range(CG // L):
        rows0[r, pl.ds(j * L, L)] = jnp.zeros((L,), jnp.float32)

    for t in range(SLAB // C):
      pltpu.sync_copy(rows0, Q.at[pl.ds(row0 + t * C, C)])

  def edges_issue(i, b):
    eb, wb, _, esem, _ = bufs[b]
    pltpu.async_copy(ipkE.at[s, i], eb, esem)
    pltpu.async_copy(wE.at[s, i], wb, esem)

  def edges_wait(i, b):
    eb, wb, _, esem, _ = bufs[b]
    pltpu.make_async_copy(ipkE.at[s, i], eb, esem).wait()
    pltpu.make_async_copy(wE.at[s, i], wb, esem).wait()

  def scale(b):
    _, wb, rows, _, _ = bufs[b]

    @pl.loop(0, C // L)
    def _(k):
      w16 = wb[pl.ds(k * L, L)]
      for e in range(L):
        we = w16[e]
        r = k * L + e
        for j in range(CG // L):
          rows[r, pl.ds(j * L, L)] = rows[r, pl.ds(j * L, L)] * we

  def spmm_pass(src_view, fix_idx):
    def idx_fix(b):
      eb, _, _, _, _ = bufs[b]
      if fix_idx:
        for k in range(C // L):
          v = eb[0, pl.ds(k * L, L)]
          eb[0, pl.ds(k * L, L)] = v * 2 + c
    def gather_issue(b):
      eb, _, rows, _, gsem = bufs[b]
      h = C // 2
      pltpu.async_copy(src_view.at[eb.at[0, pl.ds(0, h)]],
                       rows.at[pl.ds(0, h)], gsem)
      pltpu.async_copy(src_view.at[eb.at[0, pl.ds(h, h)]],
                       rows.at[pl.ds(h, h)], gsem)

    def gather_wait(b):
      eb, _, rows, _, gsem = bufs[b]
      h = C // 2
      pltpu.make_async_copy(src_view.at[eb.at[0, pl.ds(0, h)]],
                            rows.at[pl.ds(0, h)], gsem).wait()
      pltpu.make_async_copy(src_view.at[eb.at[0, pl.ds(h, h)]],
                            rows.at[pl.ds(h, h)], gsem).wait()

    def scatter_wait(b):
      rows = bufs[b][2]
      pltpu.make_async_copy(rows, Q.at[dcps[b]], ssems[b]).wait()

    def step(i, b, first, next_gather, next_edges):
      eb, _, rows, _, _ = bufs[b]
      gather_wait(b)
      if next_gather:
        edges_wait(i + 1, 1 - b)
        idx_fix(1 - b)
        if not first:
          scatter_wait(1 - b)
        gather_issue(1 - b)
      scale(b)
      for k in range(C // L):
        dcps[b][pl.ds(k * L, L)] = eb[1, pl.ds(k * L, L)]
      pltpu.async_copy(rows, Q.at[dcps[b]], ssems[b], add=True)
      if next_edges:
        edges_issue(i + 2, b)

    edges_issue(0, 0)
    edges_wait(0, 0)
    idx_fix(0)
    gather_issue(0)
    edges_issue(1, 1)
    step(0, 0, True, True, True)
    step(1, 1, False, True, True)

    @pl.loop(1, NCH // 2 - 1)
    def _(t):
      i0 = 2 * t
      step(i0, 0, False, True, True)
      step(i0 + 1, 1, False, True, True)

    step(NCH - 2, 0, False, True, False)
    step(NCH - 1, 1, False, False, False)
    scatter_wait(0)
    scatter_wait(1)

  zero_slab()
  plsc.subcore_barrier()

  spmm_pass(xV, True)
  plsc.subcore_barrier()

  pltpu.sync_copy(Q.at[pl.ds(row0, SLAB)], y1T.at[c, pl.ds(row0, SLAB)])
  zero_slab()
  plsc.subcore_barrier()

  spmm_pass(y1T.at[c], False)
  plsc.subcore_barrier()

  pltpu.sync_copy(Q.at[pl.ds(row0, SLAB)], s2T.at[c, pl.ds(row0, SLAB)])


def _make_spmm(interpret=False):
  return pl.kernel(
      _spmm_body,
      out_type=(
          jax.ShapeDtypeStruct((NC, NROW, CG), jnp.float32),
          jax.ShapeDtypeStruct((NC, NROW, CG), jnp.float32),
      ),
      mesh=plsc.VectorSubcoreMesh(
          core_axis_name="c", subcore_axis_name="s",
          num_cores=NC, num_subcores=NS),
      scratch_types=[
          pltpu.VMEM_SHARED((NROW, CG), jnp.float32),
          pltpu.VMEM((2, C), jnp.int32),
          pltpu.VMEM((C,), jnp.float32),
          pltpu.VMEM((C, CG), jnp.float32),
          pltpu.VMEM((C,), jnp.int32),
          pltpu.VMEM((2, C), jnp.int32),
          pltpu.VMEM((C,), jnp.float32),
          pltpu.VMEM((C, CG), jnp.float32),
          pltpu.VMEM((C,), jnp.int32),
          pltpu.SemaphoreType.DMA,
          pltpu.SemaphoreType.DMA,
          pltpu.SemaphoreType.DMA,
          pltpu.SemaphoreType.DMA,
          pltpu.SemaphoreType.DMA,
          pltpu.SemaphoreType.DMA,
      ],
      interpret=interpret,
  )


_spmm = _make_spmm()


MMR = 400


def _mm_body(x0b, y1b, s2b, w0, w1, w2, bb, ob):
  a = jnp.dot(x0b[...], w0[...] - w2[...], preferred_element_type=jnp.float32)
  acc2 = None
  for q in range(NC):
    w1s = w1[q * CG:(q + 1) * CG, :]
    w2s = w2[q * CG:(q + 1) * CG, :]
    a = a + jnp.dot(y1b[q], w1s, preferred_element_type=jnp.float32)
    d2 = jnp.dot(s2b[q], w2s, preferred_element_type=jnp.float32)
    acc2 = d2 if acc2 is None else acc2 + d2
  ob[...] = a + 2.0 * acc2 + bb[...]


def _make_mm(interpret=False):
  return pl.pallas_call(
      _mm_body,
      grid=(N // MMR,),
      in_specs=[
          pl.BlockSpec((MMR, D), lambda i: (i, 0)),
          pl.BlockSpec((NC, MMR, CG), lambda i: (0, i, 0)),
          pl.BlockSpec((NC, MMR, CG), lambda i: (0, i, 0)),
          pl.BlockSpec((D, OUT), lambda i: (0, 0)),
          pl.BlockSpec((D, OUT), lambda i: (0, 0)),
          pl.BlockSpec((D, OUT), lambda i: (0, 0)),
          pl.BlockSpec((1, OUT), lambda i: (0, 0)),
      ],
      out_specs=pl.BlockSpec((MMR, OUT), lambda i: (i, 0)),
      out_shape=jax.ShapeDtypeStruct((N, OUT), jnp.float32),
      interpret=interpret,
  )


_mm = _make_mm()


@jax.jit
def kernel(features, edge_index, edge_weight, W, bias):
  x0 = features.reshape(N, D)
  xV = x0.reshape(N * NC, CG)

  pad = E_PAD - E
  iot = jnp.arange(pad, dtype=jnp.int32)
  srcp = jnp.concatenate([edge_index[0], iot % N]).reshape(NS, NCH, C)
  dstp = jnp.concatenate([edge_index[1], N + iot % (NROW - N)]).reshape(
      NS, NCH, C)
  wp = jnp.concatenate(
      [edge_weight, jnp.zeros((pad,), jnp.float32)]).reshape(NS, NCH, C)
  ipk = jnp.stack([srcp, dstp], axis=2)
  y1T, s2T = _spmm(xV, ipk, wp)

  W3 = W.reshape(D, 3, OUT)
  out = _mm(x0, y1T, s2T, W3[:, 0, :], W3[:, 1, :], W3[:, 2, :],
            bias.reshape(1, OUT))
  return out.reshape(1, N, OUT)
